# baseline (speedup 1.0000x reference)
import functools
import math

import jax
import jax.numpy as jnp
from jax import lax
from jax.experimental import pallas as pl
from jax.experimental.pallas import tpu as pltpu
from jax.experimental.pallas import tpu_sc as plsc


_SCALE = math.sqrt(4096.0)

_B, _S, _D = 4, 2048, 4096
_NW = 32
_ROWS_PER_W = _S // _NW
_SUB_ROWS = 8
_CHUNK = _SUB_ROWS * _D
_N_SUB = _ROWS_PER_W // _SUB_ROWS


def _compute_chunk(x_buf, pos_buf):
    def body(j, _):
        base = j * 128
        for k in range(8):
            s = base + k * 16
            x_buf[pl.ds(s, 16)] = (
                x_buf[pl.ds(s, 16)] * _SCALE + pos_buf[pl.ds(s, 16)]
            )
        return 0

    lax.fori_loop(0, _CHUNK // 128, body, 0)


@functools.partial(
    pl.kernel,
    mesh=plsc.VectorSubcoreMesh(core_axis_name="c", subcore_axis_name="s"),
    out_type=jax.ShapeDtypeStruct((_B, _S * _D), jnp.float32),
    scratch_types=[
        pltpu.VMEM((_CHUNK,), jnp.float32),
        pltpu.VMEM((_CHUNK,), jnp.float32),
    ],
)
def _sc_add_pos(x_hbm, pos_hbm, out_hbm, x_buf, pos_buf):
    wid = lax.axis_index("s") * 2 + lax.axis_index("c")
    for sub in range(_N_SUB):
        off = (wid * _ROWS_PER_W + sub * _SUB_ROWS) * _D
        pltpu.sync_copy(pos_hbm.at[pl.ds(off, _CHUNK)], pos_buf)
        for b in range(_B):
            pltpu.sync_copy(x_hbm.at[b, pl.ds(off, _CHUNK)], x_buf)
            _compute_chunk(x_buf, pos_buf)
            pltpu.sync_copy(x_buf, out_hbm.at[b, pl.ds(off, _CHUNK)])


@jax.jit
def kernel(inputs, pos_table):
    b, s, d = inputs.shape
    out = _sc_add_pos(inputs.reshape(b, s * d), pos_table.reshape(s * d))
    return out.reshape(b, s, d)


# speedup vs baseline: 1.2343x; 1.2343x over previous
import functools
import math

import jax
import jax.numpy as jnp
from jax import lax
from jax.experimental import pallas as pl
from jax.experimental.pallas import tpu as pltpu
from jax.experimental.pallas import tpu_sc as plsc


_SCALE = math.sqrt(4096.0)

_B, _S, _D = 4, 2048, 4096
_NW = 32
_ROWS_PER_W = _S // _NW
_SUB_ROWS = 8
_CHUNK = _SUB_ROWS * _D
_N_SUB = _ROWS_PER_W // _SUB_ROWS


def _compute_chunk(x_buf, pos_buf):
    def body(j, _):
        base = j * 128
        for k in range(8):
            s = base + k * 16
            x_buf[pl.ds(s, 16)] = (
                x_buf[pl.ds(s, 16)] * _SCALE + pos_buf[pl.ds(s, 16)]
            )
        return 0

    lax.fori_loop(0, _CHUNK // 128, body, 0)


@functools.partial(
    pl.kernel,
    mesh=plsc.VectorSubcoreMesh(core_axis_name="c", subcore_axis_name="s"),
    out_type=jax.ShapeDtypeStruct((_B, _S * _D), jnp.float32),
    scratch_types=[
        pltpu.VMEM((_CHUNK,), jnp.float32),
        pltpu.VMEM((_CHUNK,), jnp.float32),
        pltpu.VMEM((_CHUNK,), jnp.float32),
        pltpu.VMEM((_CHUNK,), jnp.float32),
        pltpu.SemaphoreType.DMA,
        pltpu.SemaphoreType.DMA,
        pltpu.SemaphoreType.DMA,
        pltpu.SemaphoreType.DMA,
        pltpu.SemaphoreType.DMA,
        pltpu.SemaphoreType.DMA,
    ],
)
def _sc_add_pos(
    x_hbm, pos_hbm, out_hbm,
    xb0, xb1, pb0, pb1,
    sl0, sl1, ss0, ss1, sp0, sp1,
):
    wid = lax.axis_index("s") * 2 + lax.axis_index("c")
    xb = (xb0, xb1)
    pb = (pb0, pb1)
    sl = (sl0, sl1)
    ss = (ss0, ss1)
    sp = (sp0, sp1)

    def chunk_off(sub):
        return (wid * _ROWS_PER_W + sub * _SUB_ROWS) * _D

    def start_x_load(i):
        sub, b = divmod(i, _B)
        p = i & 1
        return pltpu.async_copy(
            x_hbm.at[b, pl.ds(chunk_off(sub), _CHUNK)], xb[p], sl[p]
        )

    def start_pos_load(sub):
        return pltpu.async_copy(
            pos_hbm.at[pl.ds(chunk_off(sub), _CHUNK)], pb[sub & 1], sp[sub & 1]
        )

    n_items = _N_SUB * _B
    pos_handles = [None] * _N_SUB
    pos_handles[0] = start_pos_load(0)
    load_handles = [None, None]
    store_handles = [None, None]
    load_handles[0] = start_x_load(0)

    for i in range(n_items):
        sub, b = divmod(i, _B)
        p = i & 1
        if b == 0:
            pos_handles[sub].wait()
            if sub + 1 < _N_SUB:
                pos_handles[sub + 1] = start_pos_load(sub + 1)
        if i + 1 < n_items:
            q = 1 - p
            if store_handles[q] is not None:
                store_handles[q].wait()
            load_handles[q] = start_x_load(i + 1)
        load_handles[p].wait()
        _compute_chunk(xb[p], pb[sub & 1])
        store_handles[p] = pltpu.async_copy(
            xb[p], out_hbm.at[b, pl.ds(chunk_off(sub), _CHUNK)], ss[p]
        )

    for h in store_handles:
        if h is not None:
            h.wait()


@jax.jit
def kernel(inputs, pos_table):
    b, s, d = inputs.shape
    out = _sc_add_pos(inputs.reshape(b, s * d), pos_table.reshape(s * d))
    return out.reshape(b, s, d)


# speedup vs baseline: 4.7848x; 3.8766x over previous
import math

import jax
import jax.numpy as jnp
from jax import lax
from jax.experimental import pallas as pl
from jax.experimental.pallas import tpu as pltpu


_SCALE = math.sqrt(4096.0)
_R = 256
_NBUF = 4


def _ring_kernel(x_hbm, pos_hbm, o_hbm, xb, posb, sem_l, sem_s, sem_p):
    b, s, d = x_hbm.shape
    ns = s // _R
    n = ns * b

    def x_copy(i, slot):
        sb = i // b
        bb = lax.rem(i, b)
        return pltpu.make_async_copy(
            x_hbm.at[bb, pl.ds(sb * _R, _R), :], xb.at[slot], sem_l.at[slot]
        )

    def store_copy(i, slot):
        sb = i // b
        bb = lax.rem(i, b)
        return pltpu.make_async_copy(
            xb.at[slot], o_hbm.at[bb, pl.ds(sb * _R, _R), :], sem_s.at[slot]
        )

    def pos_copy(sb):
        par = lax.rem(sb, 2)
        return pltpu.make_async_copy(
            pos_hbm.at[pl.ds(sb * _R, _R), :], posb.at[par], sem_p.at[par]
        )

    pos_copy(0).start()
    for i in range(_NBUF - 1):
        x_copy(i, i).start()

    def body(i, _):
        slot = lax.rem(i, _NBUF)
        sb = i // b
        bb = lax.rem(i, b)

        @pl.when(bb == 0)
        def _():
            pos_copy(sb).wait()

            @pl.when(sb + 1 < ns)
            def _():
                pos_copy(sb + 1).start()

        x_copy(i, slot).wait()
        xv = xb[slot]
        pv = posb[lax.rem(sb, 2)]
        xb[slot] = xv * _SCALE + pv
        store_copy(i, slot).start()

        j = i + _NBUF - 1

        @pl.when(j < n)
        def _():
            jslot = lax.rem(j, _NBUF)

            @pl.when(i >= 1)
            def _():
                store_copy(j - _NBUF, jslot).wait()

            x_copy(j, jslot).start()

        return 0

    lax.fori_loop(0, n, body, 0)

    def drain(i, _):
        slot = lax.rem(i, _NBUF)
        store_copy(i, slot).wait()
        return 0

    lax.fori_loop(n - _NBUF, n, drain, 0)


@jax.jit
def kernel(inputs, pos_table):
    b, s, d = inputs.shape
    return pl.pallas_call(
        _ring_kernel,
        in_specs=[
            pl.BlockSpec(memory_space=pl.ANY),
            pl.BlockSpec(memory_space=pl.ANY),
        ],
        out_specs=pl.BlockSpec(memory_space=pl.ANY),
        out_shape=jax.ShapeDtypeStruct((b, s, d), inputs.dtype),
        scratch_shapes=[
            pltpu.VMEM((_NBUF, _R, d), jnp.float32),
            pltpu.VMEM((2, _R, d), jnp.float32),
            pltpu.SemaphoreType.DMA((_NBUF,)),
            pltpu.SemaphoreType.DMA((_NBUF,)),
            pltpu.SemaphoreType.DMA((2,)),
        ],
    )(inputs, pos_table)
